# Initial kernel scaffold; baseline (speedup 1.0000x reference)
import jax
import jax.numpy as jnp
from jax.experimental import pallas as pl


def kernel(input_ids, table, pe):
    raise NotImplementedError("write your pallas kernel here")


# speedup vs baseline: 2.1187x; 2.1187x over previous
import functools

import jax
import jax.numpy as jnp
from jax import lax
from jax.experimental import pallas as pl
from jax.experimental.pallas import tpu as pltpu
from jax.experimental.pallas import tpu_sc as plsc

VOCAB = 100000
EMB = 128
B = 4096
S = 200

_NC = 2
_NS = 16
_NW = _NC * _NS

_SEQ_PER_W = B // _NW
_HALF = S // 2


def _sc_kernel(ids_hbm, table_hbm, pe_hbm, out_hbm, idx_v, pe_v, rows_v, sem):
    wid = lax.axis_index("s") * _NC + lax.axis_index("c")

    pltpu.sync_copy(pe_hbm, pe_v)

    def seq_body(t, carry):
        seq = wid * _SEQ_PER_W + t
        pltpu.sync_copy(ids_hbm.at[pl.ds(2 * seq, 2)], idx_v)
        cp0 = pltpu.make_async_copy(table_hbm.at[idx_v.at[0]],
                                    rows_v.at[pl.ds(0, _HALF)], sem)
        cp1 = pltpu.make_async_copy(table_hbm.at[idx_v.at[1]],
                                    rows_v.at[pl.ds(_HALF, _HALF)], sem)
        cp0.start()
        cp1.start()
        cp0.wait()
        cp1.wait()

        def add_body(i, c):
            for j in range(EMB // 16):
                sl = pl.ds(j * 16, 16)
                rows_v[i, sl] = rows_v[i, sl] + pe_v[i, sl]
            return c

        lax.fori_loop(0, S, add_body, 0, unroll=2)

        pltpu.sync_copy(rows_v, out_hbm.at[pl.ds(seq * S, S)])
        return carry

    lax.fori_loop(0, _SEQ_PER_W, seq_body, 0)


@functools.partial(jax.jit, static_argnames=())
def kernel(input_ids, table, pe):
    ids_flat = input_ids.reshape(2 * B, _HALF).astype(jnp.int32)
    pe_s = pe[:S, :]
    mesh = plsc.VectorSubcoreMesh(core_axis_name="c", subcore_axis_name="s")
    out = pl.kernel(
        _sc_kernel,
        mesh=mesh,
        out_type=jax.ShapeDtypeStruct((B * S, EMB), jnp.float32),
        scratch_types=[
            pltpu.VMEM((2, _HALF), jnp.int32),
            pltpu.VMEM((S, EMB), jnp.float32),
            pltpu.VMEM((S, EMB), jnp.float32),
            pltpu.SemaphoreType.DMA,
        ],
    )(ids_flat, table, pe_s)
    return out.reshape(B, S, EMB)
